# baseline (speedup 1.0000x reference)
import jax
import jax.numpy as jnp
from jax.experimental import pallas as pl

def _body(a0_ref, a1_ref, out_ref):
    out_ref[:200] = jnp.sum(a0_ref[...], axis=1, keepdims=True)
    out_ref[200:] = jnp.sum(a1_ref[...], axis=1, keepdims=True)

def kernel(x, adj, W, a, b):
    N = adj.shape[0]
    BR = 200
    return pl.pallas_call(
        _body,
        grid=(N // (2 * BR),),
        in_specs=[pl.BlockSpec((BR, N), lambda i: (2 * i, 0)),
                  pl.BlockSpec((BR, N), lambda i: (2 * i + 1, 0))],
        out_specs=pl.BlockSpec((2 * BR, 1), lambda i: (i, 0)),
        out_shape=jax.ShapeDtypeStruct((N, 1), jnp.float32),
    )(adj, adj)
